# Initial kernel scaffold; baseline (speedup 1.0000x reference)
import jax
import jax.numpy as jnp
from jax.experimental import pallas as pl
from jax.experimental.pallas import tpu as pltpu

B = 8
S = 2048
T = 65
E = 2048
ROWS_OUT = S + T
RB = 128
NBLK = (ROWS_OUT + RB - 1) // RB


def _dit_kernel(ts_ref, mask_ref, na_ref, npos_ref, tpos_ref,
                w1_ref, b1_ref, w2_ref, b2_ref, wt_ref,
                wt1_ref, bt1_ref, wt2_ref, bt2_ref,
                dit_ref, vl_ref):
    b = pl.program_id(0)
    vl_ref[0] = jnp.sum(mask_ref[...])
    t = ts_ref[b].astype(jnp.float32)
    x = t * wt_ref[...]
    x = jnp.concatenate([jnp.cos(x), jnp.sin(x)], axis=1)
    h1 = x @ wt1_ref[...] + bt1_ref[...]
    h1 = h1 * jax.nn.sigmoid(h1)
    tt = h1 @ wt2_ref[...] + bt2_ref[...] + tpos_ref[...]
    a = na_ref[...]
    g = a @ w1_ref[...] + b1_ref[...]
    g = jax.nn.gelu(g, approximate=True)
    h = g @ w2_ref[...] + b2_ref[...] + npos_ref[...]
    dit_ref[...] = jnp.concatenate([tt, h], axis=0)


def _place_kernel(vl_sm, in_ref, dit_ref, out_ref):
    b = pl.program_id(0)
    i = pl.program_id(1)
    vl = vl_sm[b]
    r0 = i * RB
    rows = r0 + jax.lax.broadcasted_iota(jnp.int32, (RB, 1), 0)
    rel = rows - vl
    in_window = (rel >= 0) & (rel < T)
    keep = jnp.logical_not(in_window) & (rows < S)
    x = jnp.where(keep, in_ref[...], 0.0)

    intersects = (vl < r0 + RB) & (vl + T > r0)

    @pl.when(intersects)
    def _():
        j = jax.lax.broadcasted_iota(jnp.int32, (RB, T), 1)
        p = ((rel == j) & in_window).astype(jnp.float32)
        win = jax.lax.dot(p, dit_ref[...],
                          preferred_element_type=jnp.float32)
        out_ref[...] = x + win

    @pl.when(jnp.logical_not(intersects))
    def _():
        out_ref[...] = x


def _compute_dit(timesteps, attention_mask, noisy_actions, noise_pos,
                 timestep_pos, W1, b1, W2, b2, w_time, Wt1, bt1, Wt2, bt2,
                 interpret=False):
    full = lambda shape: pl.BlockSpec(shape, lambda b: (0,) * len(shape))
    grid_spec = pltpu.PrefetchScalarGridSpec(
        num_scalar_prefetch=0,
        grid=(B,),
        in_specs=[
            pl.BlockSpec(memory_space=pltpu.SMEM),
            pl.BlockSpec((1, S), lambda b: (b, 0)),
            pl.BlockSpec((None, 64, 32), lambda b: (b, 0, 0)),
            full((64, E)),
            full((1, E)),
            full((32, 32)), full((1, 32)),
            full((32, E)), full((1, E)),
            full((1, 128)),
            full((256, E)), full((1, E)),
            full((E, E)), full((1, E)),
        ],
        out_specs=[
            pl.BlockSpec((None, T, E), lambda b: (b, 0, 0)),
            pl.BlockSpec((1,), lambda b: (b,), memory_space=pltpu.SMEM),
        ],
    )
    return pl.pallas_call(
        _dit_kernel,
        grid_spec=grid_spec,
        out_shape=[
            jax.ShapeDtypeStruct((B, T, E), jnp.float32),
            jax.ShapeDtypeStruct((B,), jnp.int32),
        ],
        interpret=interpret,
    )(timesteps, attention_mask, noisy_actions,
      noise_pos.reshape(64, E), timestep_pos.reshape(1, E),
      W1, b1.reshape(1, 32), W2, b2.reshape(1, E),
      w_time.reshape(1, 128), Wt1, bt1.reshape(1, E), Wt2, bt2.reshape(1, E))


def _place(vl, inputs_embeds, dit, interpret=False):
    grid_spec = pltpu.PrefetchScalarGridSpec(
        num_scalar_prefetch=1,
        grid=(B, NBLK),
        in_specs=[
            pl.BlockSpec((None, RB, E),
                         lambda vl_sm, b, i: (b, jnp.minimum(i, S // RB - 1), 0)),
            pl.BlockSpec((None, T, E), lambda vl_sm, b, i: (b, 0, 0)),
        ],
        out_specs=pl.BlockSpec((None, RB, E), lambda vl_sm, b, i: (b, i, 0)),
    )
    return pl.pallas_call(
        _place_kernel,
        grid_spec=grid_spec,
        out_shape=jax.ShapeDtypeStruct((B, ROWS_OUT, E), jnp.float32),
        interpret=interpret,
    )(vl, inputs_embeds, dit)


def kernel(noisy_actions, timesteps, input_ids, attention_mask, inputs_embeds,
           noise_pos, timestep_pos, W1, b1, W2, b2, w_time, Wt1, bt1, Wt2,
           bt2):
    dit, vl = _compute_dit(timesteps, attention_mask, noisy_actions,
                           noise_pos, timestep_pos, W1, b1, W2, b2, w_time,
                           Wt1, bt1, Wt2, bt2)
    return _place(vl, inputs_embeds, dit)


# speedup vs baseline: 1.0530x; 1.0530x over previous
import jax
import jax.numpy as jnp
from jax.experimental import pallas as pl
from jax.experimental.pallas import tpu as pltpu

B = 8
S = 2048
T = 65
E = 2048
ROWS_OUT = S + T
RB = 128
NBLK = (ROWS_OUT + RB - 1) // RB


def _dit_kernel(ts_ref, mask_ref, na_ref, npos_ref, tpos_ref,
                w1_ref, b1_ref, w2_ref, b2_ref, wt_ref,
                wt1_ref, bt1_ref, wt2_ref, bt2_ref,
                dit_ref, vl_ref):
    b = pl.program_id(0)
    vl_ref[b] = jnp.sum(mask_ref[...])
    t = ts_ref[b].astype(jnp.float32)
    x = t * wt_ref[...]
    x = jnp.concatenate([jnp.cos(x), jnp.sin(x)], axis=1)
    h1 = x @ wt1_ref[...] + bt1_ref[...]
    h1 = h1 * jax.nn.sigmoid(h1)
    tt = h1 @ wt2_ref[...] + bt2_ref[...] + tpos_ref[...]
    a = na_ref[...]
    g = a @ w1_ref[...] + b1_ref[...]
    g = jax.nn.gelu(g, approximate=True)
    h = g @ w2_ref[...] + b2_ref[...] + npos_ref[...]
    dit_ref[...] = jnp.concatenate([tt, h], axis=0)


def _place_kernel(vl_sm, in_ref, dit_ref, out_ref):
    b = pl.program_id(0)
    i = pl.program_id(1)
    vl = vl_sm[b]
    r0 = i * RB
    rows = r0 + jax.lax.broadcasted_iota(jnp.int32, (RB, 1), 0)
    rel = rows - vl
    in_window = (rel >= 0) & (rel < T)
    keep = jnp.logical_not(in_window) & (rows < S)
    x = jnp.where(keep, in_ref[...], 0.0)

    intersects = (vl < r0 + RB) & (vl + T > r0)

    @pl.when(intersects)
    def _():
        j = jax.lax.broadcasted_iota(jnp.int32, (RB, T), 1)
        p = ((rel == j) & in_window).astype(jnp.float32)
        win = jax.lax.dot(p, dit_ref[...],
                          preferred_element_type=jnp.float32)
        out_ref[...] = x + win

    @pl.when(jnp.logical_not(intersects))
    def _():
        out_ref[...] = x


def _compute_dit(timesteps, attention_mask, noisy_actions, noise_pos,
                 timestep_pos, W1, b1, W2, b2, w_time, Wt1, bt1, Wt2, bt2,
                 interpret=False):
    full = lambda shape: pl.BlockSpec(shape, lambda b: (0,) * len(shape))
    grid_spec = pltpu.PrefetchScalarGridSpec(
        num_scalar_prefetch=0,
        grid=(B,),
        in_specs=[
            pl.BlockSpec(memory_space=pltpu.SMEM),
            pl.BlockSpec((None, 1, S), lambda b: (b, 0, 0)),
            pl.BlockSpec((None, 64, 32), lambda b: (b, 0, 0)),
            full((64, E)),
            full((1, E)),
            full((32, 32)), full((1, 32)),
            full((32, E)), full((1, E)),
            full((1, 128)),
            full((256, E)), full((1, E)),
            full((E, E)), full((1, E)),
        ],
        out_specs=[
            pl.BlockSpec((None, T, E), lambda b: (b, 0, 0)),
            pl.BlockSpec((B,), lambda b: (0,), memory_space=pltpu.SMEM),
        ],
    )
    return pl.pallas_call(
        _dit_kernel,
        grid_spec=grid_spec,
        out_shape=[
            jax.ShapeDtypeStruct((B, T, E), jnp.float32),
            jax.ShapeDtypeStruct((B,), jnp.int32),
        ],
        interpret=interpret,
    )(timesteps, attention_mask.reshape(B, 1, S), noisy_actions,
      noise_pos.reshape(64, E), timestep_pos.reshape(1, E),
      W1, b1.reshape(1, 32), W2, b2.reshape(1, E),
      w_time.reshape(1, 128), Wt1, bt1.reshape(1, E), Wt2, bt2.reshape(1, E))


def _place(vl, inputs_embeds, dit, interpret=False):
    grid_spec = pltpu.PrefetchScalarGridSpec(
        num_scalar_prefetch=1,
        grid=(B, NBLK),
        in_specs=[
            pl.BlockSpec((None, RB, E),
                         lambda b, i, vl_sm: (b, jax.lax.min(i, S // RB - 1), 0)),
            pl.BlockSpec((None, T, E), lambda b, i, vl_sm: (b, 0, 0)),
        ],
        out_specs=pl.BlockSpec((None, RB, E), lambda b, i, vl_sm: (b, i, 0)),
    )
    return pl.pallas_call(
        _place_kernel,
        grid_spec=grid_spec,
        out_shape=jax.ShapeDtypeStruct((B, ROWS_OUT, E), jnp.float32),
        interpret=interpret,
    )(vl, inputs_embeds, dit)


def kernel(noisy_actions, timesteps, input_ids, attention_mask, inputs_embeds,
           noise_pos, timestep_pos, W1, b1, W2, b2, w_time, Wt1, bt1, Wt2,
           bt2):
    dit, vl = _compute_dit(timesteps, attention_mask, noisy_actions,
                           noise_pos, timestep_pos, W1, b1, W2, b2, w_time,
                           Wt1, bt1, Wt2, bt2)
    return _place(vl, inputs_embeds, dit)
